# baseline (speedup 1.0000x reference)
import functools

import jax
import jax.numpy as jnp
from jax import lax
from jax.experimental import pallas as pl
from jax.experimental.pallas import tpu as pltpu
from jax.experimental.pallas import tpu_sc as plsc

N = 10000
E = 320000
NC = 2
NS = 16
NW = NC * NS
CH = 112
CPT = 90
EPT = CH * CPT
E_PAD = EPT * NW
N_PAD = 10240
RPT = N_PAD // NS
NBUF = 3
NITER = CPT // NBUF
BN = 512


def _qmat_body(x_ref, qt_ref, b_ref, o_ref):
    o_ref[...] = jnp.maximum(
        jnp.dot(x_ref[...], qt_ref[...], preferred_element_type=jnp.float32)
        + b_ref[...], 0.0)


def _qmat(x, qt, b2):
    return pl.pallas_call(
        _qmat_body,
        grid=(N_PAD // BN,),
        in_specs=[
            pl.BlockSpec((BN, 128), lambda i: (i, 0)),
            pl.BlockSpec((128, 128), lambda i: (0, 0)),
            pl.BlockSpec((1, 128), lambda i: (0, 0)),
        ],
        out_specs=pl.BlockSpec((BN, 128), lambda i: (i, 0)),
        out_shape=jax.ShapeDtypeStruct((N_PAD, 128), jnp.float32),
    )(x, qt, b2)


def _normed_combine(a, wsv, hp, wat, wht, b):
    scale = 1.0 / jnp.maximum(wsv[0] + wsv[1], 1.0)
    agg = (a[0] + a[1]) * scale
    z = (jnp.dot(agg, wat[...], preferred_element_type=jnp.float32)
         + jnp.dot(hp[...], wht[...], preferred_element_type=jnp.float32)
         + b[...])
    z = jnp.maximum(z, 0.0)
    nrm = jnp.sqrt(jnp.sum(z * z, axis=1, keepdims=True))
    nrm = jnp.where(nrm == 0.0, 1.0, nrm)
    return z / nrm


def _combine_body(a, wsv, hp, wat, wht, b, o):
    o[...] = _normed_combine(a, wsv, hp, wat, wht, b)


def _combine_q_body(a, wsv, hp, wat, wht, b, qt, qb, oh, on):
    hn = _normed_combine(a, wsv, hp, wat, wht, b)
    oh[...] = hn
    on[...] = jnp.maximum(
        jnp.dot(hn, qt[...], preferred_element_type=jnp.float32) + qb[...],
        0.0)


_COMBINE_SPECS = [
    pl.BlockSpec((NC, BN, 128), lambda i: (0, i, 0)),
    pl.BlockSpec((NC, BN, 1), lambda i: (0, i, 0)),
    pl.BlockSpec((BN, 128), lambda i: (i, 0)),
    pl.BlockSpec((128, 128), lambda i: (0, 0)),
    pl.BlockSpec((128, 128), lambda i: (0, 0)),
    pl.BlockSpec((1, 128), lambda i: (0, 0)),
]


def _combine(a, wsv, hp, wat, wht, b2):
    return pl.pallas_call(
        _combine_body,
        grid=(N_PAD // BN,),
        in_specs=_COMBINE_SPECS,
        out_specs=pl.BlockSpec((BN, 128), lambda i: (i, 0)),
        out_shape=jax.ShapeDtypeStruct((N_PAD, 128), jnp.float32),
    )(a, wsv, hp, wat, wht, b2)


def _combine_q(a, wsv, hp, wat, wht, b2, qt, qb):
    return pl.pallas_call(
        _combine_q_body,
        grid=(N_PAD // BN,),
        in_specs=_COMBINE_SPECS + [
            pl.BlockSpec((128, 128), lambda i: (0, 0)),
            pl.BlockSpec((1, 128), lambda i: (0, 0)),
        ],
        out_specs=[pl.BlockSpec((BN, 128), lambda i: (i, 0)),
                   pl.BlockSpec((BN, 128), lambda i: (i, 0))],
        out_shape=[jax.ShapeDtypeStruct((N_PAD, 128), jnp.float32),
                   jax.ShapeDtypeStruct((N_PAD, 128), jnp.float32)],
    )(a, wsv, hp, wat, wht, b2, qt, qb)


_GATHER_DNUMS = lax.GatherDimensionNumbers(
    offset_dims=(), collapsed_slice_dims=(0,), start_index_map=(0,))


def _lane_splat(vec, j):
    idx = jnp.full((16, 1), j, jnp.int32)
    return lax.gather(vec, idx, _GATHER_DNUMS, (1,),
                      mode=lax.GatherScatterMode.PROMISE_IN_BOUNDS)


def _sc_body(table, srcs, dsts, ws, agg_out, ws_out,
             srcb, dstb, wb, srcb1, dstb1, wb1, rows0, rows1, rows2,
             agg_sp, ws_sp, isems, gsems, ssems, wsems):
    srcba = (srcb, srcb1)
    dstba = (dstb, dstb1)
    wba = (wb, wb1)
    cid = lax.axis_index("c")
    sid = lax.axis_index("s")
    wid = cid * NS + sid
    row0 = sid * RPT
    bufs = (rows0, rows1, rows2)

    zero16 = jnp.zeros((16,), jnp.float32)

    def _zrow(r, c):
        for k in range(8):
            rows0[r, pl.ds(k * 16, 16)] = zero16
        return c

    lax.fori_loop(0, CH, _zrow, 0)

    for j in range(RPT // 80):
        pltpu.sync_copy(rows0.at[pl.ds(0, 80)],
                        agg_sp.at[pl.ds(row0 + j * 80, 80)])
    for j in range(RPT // 128):
        pltpu.sync_copy(rows0.at[0], ws_sp.at[pl.ds(row0 + j * 128, 128)])
    plsc.subcore_barrier()

    def _fire_idx(g, s, b):
        pltpu.async_copy(srcs.at[wid, g], srcba[s].at[b],
                         isems.at[s * NBUF + b])
        pltpu.async_copy(dsts.at[wid, g], dstba[s].at[b],
                         isems.at[s * NBUF + b])
        pltpu.async_copy(ws.at[wid, g], wba[s].at[b], isems.at[s * NBUF + b])

    def _wait_idx(g, s, b):
        pltpu.make_async_copy(srcs.at[wid, g], srcba[s].at[b],
                              isems.at[s * NBUF + b]).wait()
        pltpu.make_async_copy(dsts.at[wid, g], dstba[s].at[b],
                              isems.at[s * NBUF + b]).wait()
        pltpu.make_async_copy(ws.at[wid, g], wba[s].at[b],
                              isems.at[s * NBUF + b]).wait()

    def _scale(buf, s, b):
        def _grp(v, c2):
            wvec = wba[s][b, pl.ds(v * 16, 16)]
            for j in range(16):
                wspl = _lane_splat(wvec, j)
                r = v * 16 + j
                for k in range(8):
                    sl = pl.ds(k * 16, 16)
                    buf[r, sl] = buf[r, sl] * wspl
            return c2

        lax.fori_loop(0, CH // 16, _grp, 0)

    def _phase(p, s):
        @pl.when(p > 0)
        def _():
            for b in range(NBUF):
                pltpu.make_async_copy(
                    bufs[b], agg_sp.at[dstba[s].at[b]], ssems.at[b]).wait()
                pltpu.make_async_copy(
                    wba[s].at[b], ws_sp.at[dstba[s].at[b]],
                    wsems.at[b]).wait()

        @pl.when(p + 1 < NITER)
        def _():
            for b in range(NBUF):
                _fire_idx((p + 1) * NBUF + b, 1 - s, b)
        for b in range(NBUF):
            g = p * NBUF + b
            _wait_idx(g, s, b)
            pltpu.async_copy(table.at[srcba[s].at[b]], bufs[b], gsems.at[b])
        for b in range(NBUF):
            pltpu.make_async_copy(table.at[srcba[s].at[b]], bufs[b],
                                  gsems.at[b]).wait()
            _scale(bufs[b], s, b)
            pltpu.async_copy(bufs[b], agg_sp.at[dstba[s].at[b]], ssems.at[b],
                             add=True)
            pltpu.async_copy(wba[s].at[b], ws_sp.at[dstba[s].at[b]],
                             wsems.at[b], add=True)

    tzero = cid * 0
    for b in range(NBUF):
        _fire_idx(tzero + b, 0, b)

    def _iter2(q, c):
        _phase(2 * q, 0)
        _phase(2 * q + 1, 1)
        return c

    lax.fori_loop(0, CPT // NBUF // 2, _iter2, 0)
    for b in range(NBUF):
        pltpu.make_async_copy(bufs[b], agg_sp.at[dstba[1].at[b]],
                              ssems.at[b]).wait()
        pltpu.make_async_copy(wba[1].at[b], ws_sp.at[dstba[1].at[b]],
                              wsems.at[b]).wait()

    plsc.subcore_barrier()

    pltpu.sync_copy(agg_sp.at[pl.ds(row0, RPT)],
                    agg_out.at[cid, pl.ds(row0, RPT)])
    pltpu.sync_copy(ws_sp.at[pl.ds(row0, RPT)],
                    ws_out.at[cid, pl.ds(row0, RPT)])


@functools.cache
def _sc_gather_scatter():
    return pl.kernel(
        _sc_body,
        out_type=[jax.ShapeDtypeStruct((NC, N_PAD, 128), jnp.float32),
                  jax.ShapeDtypeStruct((NC, N_PAD), jnp.float32)],
        mesh=plsc.VectorSubcoreMesh(core_axis_name="c", subcore_axis_name="s",
                                    num_cores=NC, num_subcores=NS),
        scratch_types=[
            pltpu.VMEM((NBUF, CH), jnp.int32),
            pltpu.VMEM((NBUF, CH), jnp.int32),
            pltpu.VMEM((NBUF, CH), jnp.float32),
            pltpu.VMEM((NBUF, CH), jnp.int32),
            pltpu.VMEM((NBUF, CH), jnp.int32),
            pltpu.VMEM((NBUF, CH), jnp.float32),
            pltpu.VMEM((CH, 128), jnp.float32),
            pltpu.VMEM((CH, 128), jnp.float32),
            pltpu.VMEM((CH, 128), jnp.float32),
            pltpu.VMEM_SHARED((N_PAD, 128), jnp.float32),
            pltpu.VMEM_SHARED((N_PAD,), jnp.float32),
            pltpu.SemaphoreType.DMA((2 * NBUF,)),
            pltpu.SemaphoreType.DMA((NBUF,)),
            pltpu.SemaphoreType.DMA((NBUF,)),
            pltpu.SemaphoreType.DMA((NBUF,)),
        ],
    )


def kernel(h, edge_index, weights, Q0_w, Q0_b, W0_w, W0_b,
           Q1_w, Q1_b, W1_w, W1_b):
    f32 = jnp.float32
    h = h.astype(f32)
    w = weights.astype(f32)
    src = edge_index[0]
    dst = edge_index[1]

    pad = E_PAD - E
    fill = (jnp.arange(pad, dtype=jnp.int32) * 37) % N
    over_i = jnp.zeros((NW, NBUF, CH), jnp.int32)
    over_f = jnp.zeros((NW, NBUF, CH), f32)

    def _tile3(x, over):
        return jnp.concatenate([x.reshape(NW, CPT, CH), over], axis=1)

    src_p = _tile3(jnp.concatenate([src, fill]), over_i)
    dst_p = _tile3(jnp.concatenate([dst, fill]), over_i)
    w_p = _tile3(jnp.concatenate([w, jnp.zeros((pad,), f32)]), over_f)

    h_pad = jnp.zeros((N_PAD, 128), f32).at[:N].set(h)

    sc = _sc_gather_scatter()
    n0 = _qmat(h_pad, Q0_w.T, Q0_b.reshape(1, 128))
    agg0, ws0 = sc(n0, src_p, dst_p, w_p)
    h1, n1 = _combine_q(agg0, ws0.reshape(NC, N_PAD, 1), h_pad,
                        W0_w[:, :128].T, W0_w[:, 128:].T,
                        W0_b.reshape(1, 128), Q1_w.T, Q1_b.reshape(1, 128))
    agg1, ws1 = sc(n1, src_p, dst_p, w_p)
    h2 = _combine(agg1, ws1.reshape(NC, N_PAD, 1), h1,
                  W1_w[:, :128].T, W1_w[:, 128:].T, W1_b.reshape(1, 128))
    return h2[:N]


# speedup vs baseline: 1.0086x; 1.0086x over previous
import functools

import jax
import jax.numpy as jnp
from jax import lax
from jax.experimental import pallas as pl
from jax.experimental.pallas import tpu as pltpu
from jax.experimental.pallas import tpu_sc as plsc

N = 10000
E = 320000
NC = 2
NS = 16
NW = NC * NS
CH = 80
CPT = 128
EPT = CH * CPT
E_PAD = EPT * NW
N_PAD = 10240
RPT = N_PAD // NS
NBUF = 4
NITER = CPT // NBUF
BN = 512


def _qmat_body(x_ref, qt_ref, b_ref, o_ref):
    o_ref[...] = jnp.maximum(
        jnp.dot(x_ref[...], qt_ref[...], preferred_element_type=jnp.float32)
        + b_ref[...], 0.0)


def _qmat(x, qt, b2):
    return pl.pallas_call(
        _qmat_body,
        grid=(N_PAD // BN,),
        in_specs=[
            pl.BlockSpec((BN, 128), lambda i: (i, 0)),
            pl.BlockSpec((128, 128), lambda i: (0, 0)),
            pl.BlockSpec((1, 128), lambda i: (0, 0)),
        ],
        out_specs=pl.BlockSpec((BN, 128), lambda i: (i, 0)),
        out_shape=jax.ShapeDtypeStruct((N_PAD, 128), jnp.float32),
    )(x, qt, b2)


def _normed_combine(a, wsv, hp, wat, wht, b):
    scale = 1.0 / jnp.maximum(wsv[0] + wsv[1], 1.0)
    agg = (a[0] + a[1]) * scale
    z = (jnp.dot(agg, wat[...], preferred_element_type=jnp.float32)
         + jnp.dot(hp[...], wht[...], preferred_element_type=jnp.float32)
         + b[...])
    z = jnp.maximum(z, 0.0)
    nrm = jnp.sqrt(jnp.sum(z * z, axis=1, keepdims=True))
    nrm = jnp.where(nrm == 0.0, 1.0, nrm)
    return z / nrm


def _combine_body(a, wsv, hp, wat, wht, b, o):
    o[...] = _normed_combine(a, wsv, hp, wat, wht, b)


def _combine_q_body(a, wsv, hp, wat, wht, b, qt, qb, oh, on):
    hn = _normed_combine(a, wsv, hp, wat, wht, b)
    oh[...] = hn
    on[...] = jnp.maximum(
        jnp.dot(hn, qt[...], preferred_element_type=jnp.float32) + qb[...],
        0.0)


_COMBINE_SPECS = [
    pl.BlockSpec((NC, BN, 128), lambda i: (0, i, 0)),
    pl.BlockSpec((NC, BN, 1), lambda i: (0, i, 0)),
    pl.BlockSpec((BN, 128), lambda i: (i, 0)),
    pl.BlockSpec((128, 128), lambda i: (0, 0)),
    pl.BlockSpec((128, 128), lambda i: (0, 0)),
    pl.BlockSpec((1, 128), lambda i: (0, 0)),
]


def _combine(a, wsv, hp, wat, wht, b2):
    return pl.pallas_call(
        _combine_body,
        grid=(N_PAD // BN,),
        in_specs=_COMBINE_SPECS,
        out_specs=pl.BlockSpec((BN, 128), lambda i: (i, 0)),
        out_shape=jax.ShapeDtypeStruct((N_PAD, 128), jnp.float32),
    )(a, wsv, hp, wat, wht, b2)


def _combine_q(a, wsv, hp, wat, wht, b2, qt, qb):
    return pl.pallas_call(
        _combine_q_body,
        grid=(N_PAD // BN,),
        in_specs=_COMBINE_SPECS + [
            pl.BlockSpec((128, 128), lambda i: (0, 0)),
            pl.BlockSpec((1, 128), lambda i: (0, 0)),
        ],
        out_specs=[pl.BlockSpec((BN, 128), lambda i: (i, 0)),
                   pl.BlockSpec((BN, 128), lambda i: (i, 0))],
        out_shape=[jax.ShapeDtypeStruct((N_PAD, 128), jnp.float32),
                   jax.ShapeDtypeStruct((N_PAD, 128), jnp.float32)],
    )(a, wsv, hp, wat, wht, b2, qt, qb)


_GATHER_DNUMS = lax.GatherDimensionNumbers(
    offset_dims=(), collapsed_slice_dims=(0,), start_index_map=(0,))


def _lane_splat(vec, j):
    idx = jnp.full((16, 1), j, jnp.int32)
    return lax.gather(vec, idx, _GATHER_DNUMS, (1,),
                      mode=lax.GatherScatterMode.PROMISE_IN_BOUNDS)


def _sc_body(table, srcs, dsts, ws, agg_out, ws_out,
             srcb, dstb, wb, srcb1, dstb1, wb1, rows0, rows1, rows2, rows3,
             agg_sp, ws_sp, isems, gsems, ssems, wsems):
    srcba = (srcb, srcb1)
    dstba = (dstb, dstb1)
    wba = (wb, wb1)
    cid = lax.axis_index("c")
    sid = lax.axis_index("s")
    wid = cid * NS + sid
    row0 = sid * RPT
    bufs = (rows0, rows1, rows2, rows3)

    zero16 = jnp.zeros((16,), jnp.float32)

    def _zrow(r, c):
        for k in range(8):
            rows0[r, pl.ds(k * 16, 16)] = zero16
        return c

    lax.fori_loop(0, CH, _zrow, 0)

    for j in range(RPT // 80):
        pltpu.sync_copy(rows0.at[pl.ds(0, 80)],
                        agg_sp.at[pl.ds(row0 + j * 80, 80)])
    for j in range(RPT // 128):
        pltpu.sync_copy(rows0.at[0], ws_sp.at[pl.ds(row0 + j * 128, 128)])
    plsc.subcore_barrier()

    def _fire_idx(g, s, b):
        pltpu.async_copy(srcs.at[wid, g], srcba[s].at[b],
                         isems.at[s * NBUF + b])
        pltpu.async_copy(dsts.at[wid, g], dstba[s].at[b],
                         isems.at[s * NBUF + b])
        pltpu.async_copy(ws.at[wid, g], wba[s].at[b], isems.at[s * NBUF + b])

    def _wait_idx(g, s, b):
        pltpu.make_async_copy(srcs.at[wid, g], srcba[s].at[b],
                              isems.at[s * NBUF + b]).wait()
        pltpu.make_async_copy(dsts.at[wid, g], dstba[s].at[b],
                              isems.at[s * NBUF + b]).wait()
        pltpu.make_async_copy(ws.at[wid, g], wba[s].at[b],
                              isems.at[s * NBUF + b]).wait()

    def _scale(buf, s, b):
        def _grp(v, c2):
            wvec = wba[s][b, pl.ds(v * 16, 16)]
            for j in range(16):
                wspl = _lane_splat(wvec, j)
                r = v * 16 + j
                for k in range(8):
                    sl = pl.ds(k * 16, 16)
                    buf[r, sl] = buf[r, sl] * wspl
            return c2

        lax.fori_loop(0, CH // 16, _grp, 0)

    def _phase(p, s):
        @pl.when(p > 0)
        def _():
            for b in range(NBUF):
                pltpu.make_async_copy(
                    bufs[b], agg_sp.at[dstba[s].at[b]], ssems.at[b]).wait()
                pltpu.make_async_copy(
                    wba[s].at[b], ws_sp.at[dstba[s].at[b]],
                    wsems.at[b]).wait()

        @pl.when(p + 1 < NITER)
        def _():
            for b in range(NBUF):
                _fire_idx((p + 1) * NBUF + b, 1 - s, b)
        for b in range(NBUF):
            g = p * NBUF + b
            _wait_idx(g, s, b)
            pltpu.async_copy(table.at[srcba[s].at[b]], bufs[b], gsems.at[b])
        for b in range(NBUF):
            pltpu.make_async_copy(table.at[srcba[s].at[b]], bufs[b],
                                  gsems.at[b]).wait()
            _scale(bufs[b], s, b)
            pltpu.async_copy(bufs[b], agg_sp.at[dstba[s].at[b]], ssems.at[b],
                             add=True)
            pltpu.async_copy(wba[s].at[b], ws_sp.at[dstba[s].at[b]],
                             wsems.at[b], add=True)

    tzero = cid * 0
    for b in range(NBUF):
        _fire_idx(tzero + b, 0, b)

    def _iter2(q, c):
        _phase(2 * q, 0)
        _phase(2 * q + 1, 1)
        return c

    lax.fori_loop(0, CPT // NBUF // 2, _iter2, 0)
    for b in range(NBUF):
        pltpu.make_async_copy(bufs[b], agg_sp.at[dstba[1].at[b]],
                              ssems.at[b]).wait()
        pltpu.make_async_copy(wba[1].at[b], ws_sp.at[dstba[1].at[b]],
                              wsems.at[b]).wait()

    plsc.subcore_barrier()

    pltpu.sync_copy(agg_sp.at[pl.ds(row0, RPT)],
                    agg_out.at[cid, pl.ds(row0, RPT)])
    pltpu.sync_copy(ws_sp.at[pl.ds(row0, RPT)],
                    ws_out.at[cid, pl.ds(row0, RPT)])


@functools.cache
def _sc_gather_scatter():
    return pl.kernel(
        _sc_body,
        out_type=[jax.ShapeDtypeStruct((NC, N_PAD, 128), jnp.float32),
                  jax.ShapeDtypeStruct((NC, N_PAD), jnp.float32)],
        mesh=plsc.VectorSubcoreMesh(core_axis_name="c", subcore_axis_name="s",
                                    num_cores=NC, num_subcores=NS),
        scratch_types=[
            pltpu.VMEM((NBUF, CH), jnp.int32),
            pltpu.VMEM((NBUF, CH), jnp.int32),
            pltpu.VMEM((NBUF, CH), jnp.float32),
            pltpu.VMEM((NBUF, CH), jnp.int32),
            pltpu.VMEM((NBUF, CH), jnp.int32),
            pltpu.VMEM((NBUF, CH), jnp.float32),
            pltpu.VMEM((CH, 128), jnp.float32),
            pltpu.VMEM((CH, 128), jnp.float32),
            pltpu.VMEM((CH, 128), jnp.float32),
            pltpu.VMEM((CH, 128), jnp.float32),
            pltpu.VMEM_SHARED((N_PAD, 128), jnp.float32),
            pltpu.VMEM_SHARED((N_PAD,), jnp.float32),
            pltpu.SemaphoreType.DMA((2 * NBUF,)),
            pltpu.SemaphoreType.DMA((NBUF,)),
            pltpu.SemaphoreType.DMA((NBUF,)),
            pltpu.SemaphoreType.DMA((NBUF,)),
        ],
    )


def kernel(h, edge_index, weights, Q0_w, Q0_b, W0_w, W0_b,
           Q1_w, Q1_b, W1_w, W1_b):
    f32 = jnp.float32
    h = h.astype(f32)
    w = weights.astype(f32)
    src = edge_index[0]
    dst = edge_index[1]

    pad = E_PAD - E
    fill = (jnp.arange(pad, dtype=jnp.int32) * 37) % N
    over_i = jnp.zeros((NW, NBUF, CH), jnp.int32)
    over_f = jnp.zeros((NW, NBUF, CH), f32)

    def _tile3(x, over):
        return jnp.concatenate([x.reshape(NW, CPT, CH), over], axis=1)

    src_p = _tile3(jnp.concatenate([src, fill]), over_i)
    dst_p = _tile3(jnp.concatenate([dst, fill]), over_i)
    w_p = _tile3(jnp.concatenate([w, jnp.zeros((pad,), f32)]), over_f)

    h_pad = jnp.zeros((N_PAD, 128), f32).at[:N].set(h)

    sc = _sc_gather_scatter()
    n0 = _qmat(h_pad, Q0_w.T, Q0_b.reshape(1, 128))
    agg0, ws0 = sc(n0, src_p, dst_p, w_p)
    h1, n1 = _combine_q(agg0, ws0.reshape(NC, N_PAD, 1), h_pad,
                        W0_w[:, :128].T, W0_w[:, 128:].T,
                        W0_b.reshape(1, 128), Q1_w.T, Q1_b.reshape(1, 128))
    agg1, ws1 = sc(n1, src_p, dst_p, w_p)
    h2 = _combine(agg1, ws1.reshape(NC, N_PAD, 1), h1,
                  W1_w[:, :128].T, W1_w[:, 128:].T, W1_b.reshape(1, 128))
    return h2[:N]
